# baseline (speedup 1.0000x reference)
import functools

import jax
import jax.numpy as jnp
from jax import lax
from jax.experimental import pallas as pl
from jax.experimental.pallas import tpu as pltpu
from jax.experimental.pallas import tpu_sc as plsc

_N = 10000
_E = 320000
_H = 128
_G = 8
_NPAD = 10112
_RPT = 632
_NW = 32
_CH = 128
_K = 1
_NHALF = 2
_HALF = 40
_NCHUNK = _NHALF * _HALF
_EPT = _K * _CH * _NCHUNK
_EPAD = _NW * _EPT
_PAD_ROWS = _NPAD - _N

@functools.cache
def _sc_kernels():
    mesh = plsc.VectorSubcoreMesh(
        core_axis_name="c", subcore_axis_name="s", num_cores=2, num_subcores=16
    )
    sc_deg = pl.kernel(
        _sc_deg_body,
        out_type=jax.ShapeDtypeStruct((2, _NPAD, _H), jnp.float32),
        mesh=mesh,
        scratch_types=[
            pltpu.VMEM((_HALF, _K * _CH), jnp.int32),
            pltpu.VMEM((_K * _CH, _H), jnp.float32),
            pltpu.SemaphoreType.DMA,
            pltpu.VMEM_SHARED((_NPAD, _H), jnp.float32),
        ],
    )
    sc_prop = pl.kernel(
        _sc_prop_body,
        out_type=jax.ShapeDtypeStruct((2, _NPAD, _H), jnp.float32),
        mesh=mesh,
        scratch_types=[
            pltpu.VMEM((_HALF, _K * _CH), jnp.int32),
            pltpu.VMEM((_HALF, _K * _CH), jnp.int32),
            pltpu.VMEM((_K * _CH, _H), jnp.float32),
            pltpu.VMEM((_K * _CH, _H), jnp.float32),
            pltpu.SemaphoreType.DMA,
            pltpu.SemaphoreType.DMA,
            pltpu.VMEM_SHARED((_NPAD, _H), jnp.float32),
        ],
    )
    return sc_deg, sc_prop


def _sc_deg_body(dst_hbm, ones_hbm, z_hbm, out_hbm, dst_v, ones_v, sem, acc_sh):
    c = lax.axis_index("c")
    s = lax.axis_index("s")
    wid = c * 16 + s
    base = s * _RPT
    pltpu.sync_copy(z_hbm, acc_sh.at[pl.ds(base, _RPT)])
    pltpu.sync_copy(ones_hbm, ones_v)
    plsc.subcore_barrier()

    _FD = 8

    def group(g, carry):
        for i in range(_FD):
            pltpu.async_copy(ones_v, acc_sh.at[dst_v.at[g * _FD + i]], sem,
                             add=True)
        for i in range(_FD):
            pltpu.make_async_copy(
                ones_v, acc_sh.at[dst_v.at[g * _FD + i]], sem
            ).wait()
        return carry

    for h in range(_NHALF):
        pltpu.sync_copy(dst_hbm.at[wid, h], dst_v)
        lax.fori_loop(0, _HALF // _FD, group, 0)
    plsc.subcore_barrier()
    pltpu.sync_copy(
        acc_sh.at[pl.ds(base, _RPT)], out_hbm.at[c, pl.ds(base, _RPT)]
    )


def _sc_prop_body(src_hbm, dst_hbm, hp_hbm, out_hbm,
                  src_v, dst_v, rows_a, rows_b, sem_a, sem_b, acc_sh):
    c = lax.axis_index("c")
    s = lax.axis_index("s")
    wid = c * 16 + s
    base = s * _RPT
    pltpu.sync_copy(hp_hbm.at[pl.ds(base, _RPT)], acc_sh.at[pl.ds(base, _RPT)])
    plsc.subcore_barrier()

    def pair(j, carry):
        pltpu.async_copy(hp_hbm.at[src_v.at[2 * j + 1]], rows_b, sem_b)
        pltpu.make_async_copy(hp_hbm.at[src_v.at[0]], rows_a, sem_a).wait()
        pltpu.sync_copy(rows_a, acc_sh.at[dst_v.at[2 * j]], add=True)

        @pl.when(j < _HALF // 2 - 1)
        def _():
            pltpu.async_copy(hp_hbm.at[src_v.at[2 * j + 2]], rows_a, sem_a)

        pltpu.make_async_copy(hp_hbm.at[src_v.at[0]], rows_b, sem_b).wait()
        pltpu.sync_copy(rows_b, acc_sh.at[dst_v.at[2 * j + 1]], add=True)
        return carry

    for h in range(_NHALF):
        pltpu.sync_copy(src_hbm.at[wid, h], src_v)
        pltpu.sync_copy(dst_hbm.at[wid, h], dst_v)
        pltpu.async_copy(hp_hbm.at[src_v.at[0]], rows_a, sem_a)
        lax.fori_loop(0, _HALF // 2, pair, 0)
    plsc.subcore_barrier()
    pltpu.sync_copy(
        acc_sh.at[pl.ds(base, _RPT)], out_hbm.at[c, pl.ds(base, _RPT)]
    )


def _bn_in(x, g, b):
    m = jnp.mean(x, axis=0, keepdims=True)
    v = jnp.mean((x - m) ** 2, axis=0, keepdims=True)
    return (x - m) / jnp.sqrt(v + 1e-5) * g + b


def _tc_pre_body(h0_ref, degp_ref, g0_ref, b0_ref, w1_ref, disp_ref, hp_ref):
    deg = degp_ref[0, : _N, :] + degp_ref[1, : _N, :] + 1.0
    dis = 1.0 / jnp.sqrt(deg)
    h = _bn_in(h0_ref[...], g0_ref[...], b0_ref[...])
    hp = dis * jnp.dot(h, w1_ref[...], preferred_element_type=jnp.float32,
                       precision=lax.Precision.HIGHEST)
    disp_ref[: _N, :] = dis
    disp_ref[_N :, :] = jnp.zeros((_PAD_ROWS, _H), jnp.float32)
    hp_ref[: _N, :] = hp
    hp_ref[_N :, :] = jnp.zeros((_PAD_ROWS, _H), jnp.float32)


_tc_pre = pl.pallas_call(
    _tc_pre_body,
    out_shape=(
        jax.ShapeDtypeStruct((_NPAD, _H), jnp.float32),
        jax.ShapeDtypeStruct((_NPAD, _H), jnp.float32),
    ),
)


def _tc_mid_body(parts_ref, hp_ref, disp_ref, b_ref, g_ref, be_ref, w_ref,
                 out_ref):
    dis = disp_ref[: _N, :]
    agg = (parts_ref[0, : _N, :] + parts_ref[1, : _N, :] - hp_ref[: _N, :]) * dis
    t = jax.nn.relu(agg + b_ref[...])
    t = _bn_in(t, g_ref[...], be_ref[...])
    nxt = dis * jnp.dot(t, w_ref[...], preferred_element_type=jnp.float32,
                        precision=lax.Precision.HIGHEST)
    out_ref[: _N, :] = nxt
    out_ref[_N :, :] = jnp.zeros((_PAD_ROWS, _H), jnp.float32)


_tc_mid = pl.pallas_call(
    _tc_mid_body,
    out_shape=jax.ShapeDtypeStruct((_NPAD, _H), jnp.float32),
)


def _tc_fin_body(parts_ref, hp_ref, disp_ref, b_ref, g_ref, be_ref, bmt_ref,
                 out_ref):
    dis = disp_ref[: _N, :]
    agg = (parts_ref[0, : _N, :] + parts_ref[1, : _N, :] - hp_ref[: _N, :]) * dis
    t = jax.nn.relu(agg + b_ref[...])
    t = _bn_in(t, g_ref[...], be_ref[...])
    bt = jnp.broadcast_to(bmt_ref[...], (_G, _N))
    gids = lax.broadcasted_iota(jnp.int32, (_G, _N), 0)
    bmt = (bt == gids).astype(jnp.float32)
    sums = jnp.dot(bmt, t, preferred_element_type=jnp.float32,
                   precision=lax.Precision.HIGHEST)
    cnt = jnp.sum(bmt, axis=1, keepdims=True)
    out_ref[...] = sums / jnp.maximum(cnt, 1.0)


_tc_fin = pl.pallas_call(
    _tc_fin_body,
    out_shape=jax.ShapeDtypeStruct((_G, _H), jnp.float32),
)


def kernel(pos, norm, x, edge_index, batch, bn0_g, bn0_b,
           W1, b1, g1, be1, W2, b2, g2, be2,
           W3, b3, g3, be3, W4, b4, g4, be4):
    src = edge_index[0]
    dst = edge_index[1]
    pad = _N + (jnp.arange(_EPAD - _E, dtype=jnp.int32) % _PAD_ROWS)
    srcp = jnp.concatenate([src, pad]).reshape(_NW, _NHALF, _HALF, _K * _CH)
    dstp = jnp.concatenate([dst, pad]).reshape(_NW, _NHALF, _HALF, _K * _CH)
    h0 = jnp.concatenate([pos, norm, x], axis=1)
    ones_rows = jnp.ones((_K * _CH, _H), jnp.float32)
    z_rows = jnp.zeros((_RPT, _H), jnp.float32)
    bt2d = batch.reshape(1, _N).astype(jnp.int32)

    sc_deg, sc_prop = _sc_kernels()
    degp = sc_deg(dstp, ones_rows, z_rows)
    disp, hp = _tc_pre(
        h0, degp, bn0_g.reshape(1, _H), bn0_b.reshape(1, _H), W1
    )
    for b_i, g_i, be_i, w_next in ((b1, g1, be1, W2),
                                   (b2, g2, be2, W3),
                                   (b3, g3, be3, W4)):
        parts = sc_prop(srcp, dstp, hp)
        hp = _tc_mid(
            parts, hp, disp,
            b_i.reshape(1, _H), g_i.reshape(1, _H), be_i.reshape(1, _H),
            w_next,
        )
    parts = sc_prop(srcp, dstp, hp)
    return _tc_fin(
        parts, hp, disp,
        b4.reshape(1, _H), g4.reshape(1, _H), be4.reshape(1, _H),
        bt2d,
    )


# speedup vs baseline: 1.0839x; 1.0839x over previous
import functools

import jax
import jax.numpy as jnp
from jax import lax
from jax.experimental import pallas as pl
from jax.experimental.pallas import tpu as pltpu
from jax.experimental.pallas import tpu_sc as plsc

_N = 10000
_E = 320000
_H = 128
_G = 8
_NPAD = 10112
_RPT = 632
_NW = 32
_CH = 128
_K = 1
_NHALF = 2
_HALF = 40
_NCHUNK = _NHALF * _HALF
_EPT = _K * _CH * _NCHUNK
_EPAD = _NW * _EPT
_PAD_ROWS = _NPAD - _N
_DW = 16

@functools.cache
def _sc_kernels():
    mesh = plsc.VectorSubcoreMesh(
        core_axis_name="c", subcore_axis_name="s", num_cores=2, num_subcores=16
    )
    sc_deg = pl.kernel(
        _sc_deg_body,
        out_type=jax.ShapeDtypeStruct((2, _NPAD, _DW), jnp.float32),
        mesh=mesh,
        scratch_types=[
            pltpu.VMEM((_HALF, _K * _CH), jnp.int32),
            pltpu.VMEM((_K * _CH, _DW), jnp.float32),
            pltpu.SemaphoreType.DMA,
            pltpu.VMEM_SHARED((_NPAD, _DW), jnp.float32),
        ],
    )
    sc_prop = pl.kernel(
        _sc_prop_body,
        out_type=jax.ShapeDtypeStruct((2, _NPAD, _H), jnp.float32),
        mesh=mesh,
        scratch_types=[
            pltpu.VMEM((_HALF, _K * _CH), jnp.int32),
            pltpu.VMEM((_HALF, _K * _CH), jnp.int32),
            pltpu.VMEM((_K * _CH, _H), jnp.float32),
            pltpu.VMEM((_K * _CH, _H), jnp.float32),
            pltpu.SemaphoreType.DMA,
            pltpu.SemaphoreType.DMA,
            pltpu.VMEM_SHARED((_NPAD, _H), jnp.float32),
        ],
    )
    return sc_deg, sc_prop


def _sc_deg_body(dst_hbm, ones_hbm, z_hbm, out_hbm, dst_v, ones_v, sem, acc_sh):
    c = lax.axis_index("c")
    s = lax.axis_index("s")
    wid = c * 16 + s
    base = s * _RPT
    pltpu.sync_copy(z_hbm, acc_sh.at[pl.ds(base, _RPT)])
    pltpu.sync_copy(ones_hbm, ones_v)
    plsc.subcore_barrier()

    _FD = 8

    def group(g, carry):
        for i in range(_FD):
            pltpu.async_copy(ones_v, acc_sh.at[dst_v.at[g * _FD + i]], sem,
                             add=True)
        for i in range(_FD):
            pltpu.make_async_copy(
                ones_v, acc_sh.at[dst_v.at[g * _FD + i]], sem
            ).wait()
        return carry

    for h in range(_NHALF):
        pltpu.sync_copy(dst_hbm.at[wid, h], dst_v)
        lax.fori_loop(0, _HALF // _FD, group, 0)
    plsc.subcore_barrier()
    pltpu.sync_copy(
        acc_sh.at[pl.ds(base, _RPT)], out_hbm.at[c, pl.ds(base, _RPT)]
    )


def _sc_prop_body(src_hbm, dst_hbm, hp_hbm, out_hbm,
                  src_v, dst_v, rows_a, rows_b, sem_a, sem_b, acc_sh):
    c = lax.axis_index("c")
    s = lax.axis_index("s")
    wid = c * 16 + s
    base = s * _RPT
    pltpu.sync_copy(hp_hbm.at[pl.ds(base, _RPT)], acc_sh.at[pl.ds(base, _RPT)])
    plsc.subcore_barrier()

    def pair(j, carry):
        pltpu.async_copy(hp_hbm.at[src_v.at[2 * j + 1]], rows_b, sem_b)
        pltpu.make_async_copy(hp_hbm.at[src_v.at[0]], rows_a, sem_a).wait()
        pltpu.sync_copy(rows_a, acc_sh.at[dst_v.at[2 * j]], add=True)

        @pl.when(j < _HALF // 2 - 1)
        def _():
            pltpu.async_copy(hp_hbm.at[src_v.at[2 * j + 2]], rows_a, sem_a)

        pltpu.make_async_copy(hp_hbm.at[src_v.at[0]], rows_b, sem_b).wait()
        pltpu.sync_copy(rows_b, acc_sh.at[dst_v.at[2 * j + 1]], add=True)
        return carry

    for h in range(_NHALF):
        pltpu.sync_copy(src_hbm.at[wid, h], src_v)
        pltpu.sync_copy(dst_hbm.at[wid, h], dst_v)
        pltpu.async_copy(hp_hbm.at[src_v.at[0]], rows_a, sem_a)
        lax.fori_loop(0, _HALF // 2, pair, 0)
    plsc.subcore_barrier()
    pltpu.sync_copy(
        acc_sh.at[pl.ds(base, _RPT)], out_hbm.at[c, pl.ds(base, _RPT)]
    )


def _bn_in(x, g, b):
    m = jnp.mean(x, axis=0, keepdims=True)
    v = jnp.mean((x - m) ** 2, axis=0, keepdims=True)
    return (x - m) / jnp.sqrt(v + 1e-5) * g + b


def _tc_pre_body(h0_ref, degp_ref, g0_ref, b0_ref, w1_ref, disp_ref, hp_ref):
    deg = degp_ref[0, : _N, 0:1] + degp_ref[1, : _N, 0:1] + 1.0
    dis = jnp.broadcast_to(1.0 / jnp.sqrt(deg), (_N, _H))
    h = _bn_in(h0_ref[...], g0_ref[...], b0_ref[...])
    hp = dis * jnp.dot(h, w1_ref[...], preferred_element_type=jnp.float32,
                       precision=lax.Precision.HIGHEST)
    disp_ref[: _N, :] = dis
    disp_ref[_N :, :] = jnp.zeros((_PAD_ROWS, _H), jnp.float32)
    hp_ref[: _N, :] = hp
    hp_ref[_N :, :] = jnp.zeros((_PAD_ROWS, _H), jnp.float32)


_tc_pre = pl.pallas_call(
    _tc_pre_body,
    out_shape=(
        jax.ShapeDtypeStruct((_NPAD, _H), jnp.float32),
        jax.ShapeDtypeStruct((_NPAD, _H), jnp.float32),
    ),
)


def _tc_mid_body(parts_ref, hp_ref, disp_ref, b_ref, g_ref, be_ref, w_ref,
                 out_ref):
    dis = disp_ref[: _N, :]
    agg = (parts_ref[0, : _N, :] + parts_ref[1, : _N, :] - hp_ref[: _N, :]) * dis
    t = jax.nn.relu(agg + b_ref[...])
    t = _bn_in(t, g_ref[...], be_ref[...])
    nxt = dis * jnp.dot(t, w_ref[...], preferred_element_type=jnp.float32,
                        precision=lax.Precision.HIGHEST)
    out_ref[: _N, :] = nxt
    out_ref[_N :, :] = jnp.zeros((_PAD_ROWS, _H), jnp.float32)


_tc_mid = pl.pallas_call(
    _tc_mid_body,
    out_shape=jax.ShapeDtypeStruct((_NPAD, _H), jnp.float32),
)


def _tc_fin_body(parts_ref, hp_ref, disp_ref, b_ref, g_ref, be_ref, bmt_ref,
                 out_ref):
    dis = disp_ref[: _N, :]
    agg = (parts_ref[0, : _N, :] + parts_ref[1, : _N, :] - hp_ref[: _N, :]) * dis
    t = jax.nn.relu(agg + b_ref[...])
    t = _bn_in(t, g_ref[...], be_ref[...])
    bt = jnp.broadcast_to(bmt_ref[...], (_G, _N))
    gids = lax.broadcasted_iota(jnp.int32, (_G, _N), 0)
    bmt = (bt == gids).astype(jnp.float32)
    sums = jnp.dot(bmt, t, preferred_element_type=jnp.float32,
                   precision=lax.Precision.HIGHEST)
    cnt = jnp.sum(bmt, axis=1, keepdims=True)
    out_ref[...] = sums / jnp.maximum(cnt, 1.0)


_tc_fin = pl.pallas_call(
    _tc_fin_body,
    out_shape=jax.ShapeDtypeStruct((_G, _H), jnp.float32),
)


def kernel(pos, norm, x, edge_index, batch, bn0_g, bn0_b,
           W1, b1, g1, be1, W2, b2, g2, be2,
           W3, b3, g3, be3, W4, b4, g4, be4):
    src = edge_index[0]
    dst = edge_index[1]
    pad = _N + (jnp.arange(_EPAD - _E, dtype=jnp.int32) % _PAD_ROWS)
    srcp = jnp.concatenate([src, pad]).reshape(_NW, _NHALF, _HALF, _K * _CH)
    dstp = jnp.concatenate([dst, pad]).reshape(_NW, _NHALF, _HALF, _K * _CH)
    h0 = jnp.concatenate([pos, norm, x], axis=1)
    ones_rows = jnp.ones((_K * _CH, _DW), jnp.float32)
    z_rows = jnp.zeros((_RPT, _DW), jnp.float32)
    bt2d = batch.reshape(1, _N).astype(jnp.int32)

    sc_deg, sc_prop = _sc_kernels()
    degp = sc_deg(dstp, ones_rows, z_rows)
    disp, hp = _tc_pre(
        h0, degp, bn0_g.reshape(1, _H), bn0_b.reshape(1, _H), W1
    )
    for b_i, g_i, be_i, w_next in ((b1, g1, be1, W2),
                                   (b2, g2, be2, W3),
                                   (b3, g3, be3, W4)):
        parts = sc_prop(srcp, dstp, hp)
        hp = _tc_mid(
            parts, hp, disp,
            b_i.reshape(1, _H), g_i.reshape(1, _H), be_i.reshape(1, _H),
            w_next,
        )
    parts = sc_prop(srcp, dstp, hp)
    return _tc_fin(
        parts, hp, disp,
        b4.reshape(1, _H), g4.reshape(1, _H), be4.reshape(1, _H),
        bt2d,
    )
